# Initial kernel scaffold; baseline (speedup 1.0000x reference)
import jax
import jax.numpy as jnp
from jax.experimental import pallas as pl


def kernel(x, edge_index, edge_attr, basis0, att0, root0, bias0, basis1, att1, root1, bias1):
    raise NotImplementedError("write your pallas kernel here")


# speedup vs baseline: 1.9592x; 1.9592x over previous
import functools

import jax
import jax.numpy as jnp
from jax import lax
from jax.experimental import pallas as pl
from jax.experimental.pallas import tpu as pltpu
from jax.experimental.pallas import tpu_sc as plsc

_N = 10000
_E = 320000
_C = 128
_R = 4
_NC = 2
_NS = 16
_L = 16

_EB = 80
_EPT = _E // _NS
_NB = _EPT // _EB
_RPT = 624
_RLAST = _N - 15 * _RPT
_ZR = 16


def _sc_edge_body(x_hbm, src_hbm, dst_hbm, ea_hbm, att_hbm, out_hbm,
                  zbuf, attv, eab, sidx, didx, rows, zsh, sem):
    c = lax.axis_index("c")
    s = lax.axis_index("s")
    zero = jnp.zeros((_L,), jnp.float32)

    @pl.loop(0, _ZR)
    def _zero_zbuf(i):
        for j in range(_C // _L):
            zbuf[i, pl.ds(j * _L, _L)] = zero

    zoff = pl.multiple_of(s * _RPT, 8)

    @pl.when(s < _NS - 1)
    def _zero_main():
        @pl.loop(0, _RPT // _ZR)
        def _(t):
            pltpu.sync_copy(zbuf, zsh.at[pl.ds(zoff + t * _ZR, _ZR)])

    @pl.when(s == _NS - 1)
    def _zero_last():
        @pl.loop(0, _RLAST // _ZR)
        def _(t):
            pltpu.sync_copy(zbuf, zsh.at[pl.ds(zoff + t * _ZR, _ZR)])

    pltpu.sync_copy(att_hbm, attv)
    a = [plsc.load_gather(attv, [jnp.full((_L,), r * _L, jnp.int32) + c])
         for r in range(_R)]

    plsc.subcore_barrier()

    @pl.loop(0, _NB)
    def _batch(i):
        base = pl.multiple_of(s * _EPT + i * _EB, 8)
        abase = pl.multiple_of((s * _NB + i) * (_R * _EB), 8)
        pltpu.sync_copy(src_hbm.at[pl.ds(base, _EB)], sidx)
        pltpu.sync_copy(dst_hbm.at[pl.ds(base, _EB)], didx)
        pltpu.sync_copy(ea_hbm.at[pl.ds(abase, _R * _EB)], eab)
        pltpu.async_copy(x_hbm.at[sidx], rows, sem).wait()

        @pl.loop(0, _EB // _L)
        def _coef(g):
            cv = a[0] * eab[pl.ds(g * _L, _L)]
            for r in range(1, _R):
                cv = cv + a[r] * eab[pl.ds(r * _EB + g * _L, _L)]
            eab[pl.ds(g * _L, _L)] = cv

        @pl.loop(0, _EB)
        def _scale(e):
            cv = plsc.load_gather(eab, [jnp.full((_L,), e, jnp.int32)])
            for j in range(_C // _L):
                rows[e, pl.ds(j * _L, _L)] = cv * rows[e, pl.ds(j * _L, _L)]

        pltpu.sync_copy(rows, zsh.at[didx], add=True)

    plsc.subcore_barrier()

    @pl.when(s < _NS - 1)
    def _out_main():
        pltpu.sync_copy(zsh.at[pl.ds(zoff, _RPT)],
                        out_hbm.at[c, pl.ds(zoff, _RPT)])

    @pl.when(s == _NS - 1)
    def _out_last():
        pltpu.sync_copy(zsh.at[pl.ds(zoff, _RLAST)],
                        out_hbm.at[c, pl.ds(zoff, _RLAST)])


_sc_edge_pass = functools.partial(
    pl.kernel,
    out_type=jax.ShapeDtypeStruct((_NC, _N, _C), jnp.float32),
    mesh=plsc.VectorSubcoreMesh(core_axis_name="c", subcore_axis_name="s"),
    compiler_params=pltpu.CompilerParams(needs_layout_passes=False),
    scratch_types=[
        pltpu.VMEM((_ZR, _C), jnp.float32),
        pltpu.VMEM((8 * _L,), jnp.float32),
        pltpu.VMEM((_R * _EB,), jnp.float32),
        pltpu.VMEM((_EB,), jnp.int32),
        pltpu.VMEM((_EB,), jnp.int32),
        pltpu.VMEM((_EB, _C), jnp.float32),
        pltpu.VMEM_SHARED((_N, _C), jnp.float32),
        pltpu.SemaphoreType.DMA,
    ],
)(_sc_edge_body)


_TCB = 1000


def _tc_update_body(relu, z_ref, x_ref, w_ref, b_ref, o_ref):
    acc = jnp.dot(z_ref[0], w_ref[0:_C], preferred_element_type=jnp.float32)
    acc = acc + jnp.dot(z_ref[1], w_ref[_C:2 * _C],
                        preferred_element_type=jnp.float32)
    acc = acc + jnp.dot(x_ref[...], w_ref[2 * _C:3 * _C],
                        preferred_element_type=jnp.float32)
    acc = acc + b_ref[...]
    o_ref[...] = jnp.maximum(acc, 0.0) if relu else acc


def _tc_update(z, xin, w, bias, relu):
    body = functools.partial(_tc_update_body, relu)
    return pl.pallas_call(
        body,
        grid=(_N // _TCB,),
        in_specs=[
            pl.BlockSpec((_NC, _TCB, _C), lambda i: (0, i, 0)),
            pl.BlockSpec((_TCB, _C), lambda i: (i, 0)),
            pl.BlockSpec((3 * _C, _C), lambda i: (0, 0)),
            pl.BlockSpec((1, _C), lambda i: (0, 0)),
        ],
        out_specs=pl.BlockSpec((_TCB, _C), lambda i: (i, 0)),
        out_shape=jax.ShapeDtypeStruct((_N, _C), jnp.float32),
    )(z, xin, w, bias)


def kernel(x, edge_index, edge_attr, basis0, att0, root0, bias0,
           basis1, att1, root1, bias1):
    src = edge_index[0]
    dst = edge_index[1]
    ea_pk = edge_attr.reshape(_E // _EB, _EB, _R).transpose(0, 2, 1).reshape(-1)

    h = x
    for basis, att, root, bias, relu in (
            (basis0, att0, root0, bias0, True),
            (basis1, att1, root1, bias1, False)):
        att_pad = jnp.zeros((8, _L), jnp.float32).at[:_R, :_NC].set(att).reshape(-1)
        w = jnp.concatenate([basis[0], basis[1], root], axis=0)
        z = _sc_edge_pass(h, src, dst, ea_pk, att_pad)
        h = _tc_update(z, h, w, bias.reshape(1, _C), relu)
    return h


# speedup vs baseline: 5.2732x; 2.6916x over previous
import functools

import jax
import jax.numpy as jnp
from jax import lax
from jax.experimental import pallas as pl
from jax.experimental.pallas import tpu as pltpu
from jax.experimental.pallas import tpu_sc as plsc

_N = 10000
_E = 320000
_C = 128
_R = 4
_NC = 2
_NS = 16
_L = 16

_EB = 80
_EPT = _E // _NS
_NB = _EPT // _EB
_RPT = 624
_RLAST = _N - 15 * _RPT
_ZR = 16


def _sc_edge_body(x_hbm, src_hbm, dst_hbm, ea_hbm, att_hbm, out_hbm,
                  zbuf, attv,
                  sidx0, sidx1, sidx2, sidx3,
                  didx0, didx1, didx2, didx3,
                  eab0, eab1, eab2, eab3,
                  rows0, rows1, zsh,
                  semi0, semi1, semi2, semi3, semg0, semg1, sems0, sems1):
    sidx = (sidx0, sidx1, sidx2, sidx3)
    didx = (didx0, didx1, didx2, didx3)
    eab = (eab0, eab1, eab2, eab3)
    rows = (rows0, rows1)
    semi = (semi0, semi1, semi2, semi3)
    semg = (semg0, semg1)
    sems = (sems0, sems1)
    c = lax.axis_index("c")
    s = lax.axis_index("s")
    zero = jnp.zeros((_L,), jnp.float32)

    @pl.loop(0, _ZR)
    def _zero_zbuf(i):
        for j in range(_C // _L):
            zbuf[i, pl.ds(j * _L, _L)] = zero

    zoff = pl.multiple_of(s * _RPT, 8)

    @pl.when(s < _NS - 1)
    def _zero_main():
        @pl.loop(0, _RPT // _ZR)
        def _(t):
            pltpu.sync_copy(zbuf, zsh.at[pl.ds(zoff + t * _ZR, _ZR)])

    @pl.when(s == _NS - 1)
    def _zero_last():
        @pl.loop(0, _RLAST // _ZR)
        def _(t):
            pltpu.sync_copy(zbuf, zsh.at[pl.ds(zoff + t * _ZR, _ZR)])

    pltpu.sync_copy(att_hbm, attv)
    a = [plsc.load_gather(attv, [jnp.full((_L,), r * _L, jnp.int32) + c])
         for r in range(_R)]

    plsc.subcore_barrier()

    def issue_in(i, b):
        base = pl.multiple_of(s * _EPT + i * _EB, 8)
        abase = pl.multiple_of((s * _NB + i) * (_R * _EB), 8)
        pltpu.async_copy(src_hbm.at[pl.ds(base, _EB)], sidx[b], semi[b])
        pltpu.async_copy(dst_hbm.at[pl.ds(base, _EB)], didx[b], semi[b])
        pltpu.async_copy(ea_hbm.at[pl.ds(abase, _R * _EB)], eab[b], semi[b])

    def wait_in(b):
        pltpu.make_async_copy(src_hbm.at[pl.ds(0, _EB)], sidx[b], semi[b]).wait()
        pltpu.make_async_copy(dst_hbm.at[pl.ds(0, _EB)], didx[b], semi[b]).wait()
        pltpu.make_async_copy(ea_hbm.at[pl.ds(0, _EB * _R)], eab[b], semi[b]).wait()

    def issue_gather(b, rb):
        pltpu.async_copy(x_hbm.at[sidx[b]], rows[rb], semg[rb])

    def wait_gather(b, rb):
        pltpu.make_async_copy(x_hbm.at[sidx[b]], rows[rb], semg[rb]).wait()

    def issue_scatter(b, rb):
        pltpu.async_copy(rows[rb], zsh.at[didx[b]], sems[rb], add=True)

    def wait_scatter(b, rb):
        pltpu.make_async_copy(rows[rb], zsh.at[didx[b]], sems[rb]).wait()

    def compute(b, rb):
        @pl.loop(0, _EB // _L)
        def _coef(g):
            cv = a[0] * eab[b][pl.ds(g * _L, _L)]
            for r in range(1, _R):
                cv = cv + a[r] * eab[b][pl.ds(r * _EB + g * _L, _L)]
            eab[b][pl.ds(g * _L, _L)] = cv

        @pl.loop(0, _EB, unroll=4)
        def _scale(e):
            cv = plsc.load_gather(eab[b], [jnp.full((_L,), e, jnp.int32)])
            for j in range(_C // _L):
                rows[rb][e, pl.ds(j * _L, _L)] = cv * rows[rb][e, pl.ds(j * _L, _L)]

    for b in range(3):
        issue_in(b, b)
    wait_in(0)
    issue_gather(0, 0)

    @pl.loop(0, _NB - 2, step=4)
    def _group(g):
        for b in range(4):
            k = g + b
            rb = b % 2
            orb = (b + 1) % 2
            wait_gather(b, rb)
            if b == 0:
                @pl.when(g > 0)
                def _():
                    wait_scatter(3, orb)
            else:
                wait_scatter(b - 1, orb)
            if b == 3:
                @pl.when(g + 6 < _NB)
                def _():
                    issue_in(k + 3, (b + 3) % 4)
            else:
                issue_in(k + 3, (b + 3) % 4)
            wait_in((b + 1) % 4)
            issue_gather((b + 1) % 4, orb)
            compute(b, rb)
            issue_scatter(b, rb)

    wait_gather(0, 0)
    wait_scatter(3, 1)
    wait_in(1)
    issue_gather(1, 1)
    compute(0, 0)
    issue_scatter(0, 0)

    wait_gather(1, 1)
    wait_scatter(0, 0)
    compute(1, 1)
    pltpu.sync_copy(rows[1], zsh.at[didx[1]], add=True)

    plsc.subcore_barrier()

    @pl.when(s < _NS - 1)
    def _out_main():
        pltpu.sync_copy(zsh.at[pl.ds(zoff, _RPT)],
                        out_hbm.at[c, pl.ds(zoff, _RPT)])

    @pl.when(s == _NS - 1)
    def _out_last():
        pltpu.sync_copy(zsh.at[pl.ds(zoff, _RLAST)],
                        out_hbm.at[c, pl.ds(zoff, _RLAST)])


_sc_edge_pass = functools.partial(
    pl.kernel,
    out_type=jax.ShapeDtypeStruct((_NC, _N, _C), jnp.float32),
    mesh=plsc.VectorSubcoreMesh(core_axis_name="c", subcore_axis_name="s"),
    compiler_params=pltpu.CompilerParams(needs_layout_passes=False),
    scratch_types=(
        [pltpu.VMEM((_ZR, _C), jnp.float32),
         pltpu.VMEM((8 * _L,), jnp.float32)]
        + [pltpu.VMEM((_EB,), jnp.int32)] * 4
        + [pltpu.VMEM((_EB,), jnp.int32)] * 4
        + [pltpu.VMEM((_R * _EB,), jnp.float32)] * 4
        + [pltpu.VMEM((_EB, _C), jnp.float32)] * 2
        + [pltpu.VMEM_SHARED((_N, _C), jnp.float32)]
        + [pltpu.SemaphoreType.DMA] * 8
    ),
)(_sc_edge_body)


_TCB = 1000


def _tc_update_body(relu, z_ref, x_ref, w_ref, b_ref, o_ref):
    acc = jnp.dot(z_ref[0], w_ref[0:_C], preferred_element_type=jnp.float32)
    acc = acc + jnp.dot(z_ref[1], w_ref[_C:2 * _C],
                        preferred_element_type=jnp.float32)
    acc = acc + jnp.dot(x_ref[...], w_ref[2 * _C:3 * _C],
                        preferred_element_type=jnp.float32)
    acc = acc + b_ref[...]
    o_ref[...] = jnp.maximum(acc, 0.0) if relu else acc


def _tc_update(z, xin, w, bias, relu):
    body = functools.partial(_tc_update_body, relu)
    return pl.pallas_call(
        body,
        grid=(_N // _TCB,),
        in_specs=[
            pl.BlockSpec((_NC, _TCB, _C), lambda i: (0, i, 0)),
            pl.BlockSpec((_TCB, _C), lambda i: (i, 0)),
            pl.BlockSpec((3 * _C, _C), lambda i: (0, 0)),
            pl.BlockSpec((1, _C), lambda i: (0, 0)),
        ],
        out_specs=pl.BlockSpec((_TCB, _C), lambda i: (i, 0)),
        out_shape=jax.ShapeDtypeStruct((_N, _C), jnp.float32),
    )(z, xin, w, bias)


def kernel(x, edge_index, edge_attr, basis0, att0, root0, bias0,
           basis1, att1, root1, bias1):
    src = edge_index[0]
    dst = edge_index[1]
    ea_pk = edge_attr.reshape(_E // _EB, _EB, _R).transpose(0, 2, 1).reshape(-1)

    h = x
    for basis, att, root, bias, relu in (
            (basis0, att0, root0, bias0, True),
            (basis1, att1, root1, bias1, False)):
        att_pad = jnp.zeros((8, _L), jnp.float32).at[:_R, :_NC].set(att).reshape(-1)
        w = jnp.concatenate([basis[0], basis[1], root], axis=0)
        z = _sc_edge_pass(h, src, dst, ea_pk, att_pad)
        h = _tc_update(z, h, w, bias.reshape(1, _C), relu)
    return h


# speedup vs baseline: 5.5595x; 1.0543x over previous
import functools

import jax
import jax.numpy as jnp
from jax import lax
from jax.experimental import pallas as pl
from jax.experimental.pallas import tpu as pltpu
from jax.experimental.pallas import tpu_sc as plsc

_N = 10000
_E = 320000
_C = 128
_R = 4
_NC = 2
_NS = 16
_L = 16

_EB = 80
_EPT = _E // _NS
_NB = _EPT // _EB
_RPT = 624
_RLAST = _N - 15 * _RPT
_ZR = 16


def _sc_edge_body(x_hbm, src_hbm, dst_hbm, ea_hbm, att_hbm, out_hbm,
                  zbuf, attv,
                  sidx0, sidx1, sidx2, sidx3,
                  didx0, didx1, didx2, didx3,
                  eab0, eab1, eab2, eab3,
                  rows0, rows1, zsh,
                  semi0, semi1, semi2, semi3, semg0, semg1, sems0, sems1):
    sidx = (sidx0, sidx1, sidx2, sidx3)
    didx = (didx0, didx1, didx2, didx3)
    eab = (eab0, eab1, eab2, eab3)
    rows = (rows0, rows1)
    semi = (semi0, semi1, semi2, semi3)
    semg = (semg0, semg1)
    sems = (sems0, sems1)
    c = lax.axis_index("c")
    s = lax.axis_index("s")
    zero = jnp.zeros((_L,), jnp.float32)

    @pl.loop(0, _ZR)
    def _zero_zbuf(i):
        for j in range(_C // _L):
            zbuf[i, pl.ds(j * _L, _L)] = zero

    zoff = pl.multiple_of(s * _RPT, 8)

    @pl.when(s < _NS - 1)
    def _zero_main():
        @pl.loop(0, _RPT // _ZR)
        def _(t):
            pltpu.sync_copy(zbuf, zsh.at[pl.ds(zoff + t * _ZR, _ZR)])

    @pl.when(s == _NS - 1)
    def _zero_last():
        @pl.loop(0, _RLAST // _ZR)
        def _(t):
            pltpu.sync_copy(zbuf, zsh.at[pl.ds(zoff + t * _ZR, _ZR)])

    pltpu.sync_copy(att_hbm, attv)
    a = [plsc.load_gather(attv, [jnp.full((_L,), r * _L, jnp.int32) + c])
         for r in range(_R)]

    plsc.subcore_barrier()

    def issue_in(i, b):
        base = pl.multiple_of(s * _EPT + i * _EB, 8)
        abase = pl.multiple_of((s * _NB + i) * (_R * _EB), 8)
        pltpu.async_copy(src_hbm.at[pl.ds(base, _EB)], sidx[b], semi[b])
        pltpu.async_copy(dst_hbm.at[pl.ds(base, _EB)], didx[b], semi[b])
        pltpu.async_copy(ea_hbm.at[pl.ds(abase, _R * _EB)], eab[b], semi[b])

    def wait_in(b):
        pltpu.make_async_copy(src_hbm.at[pl.ds(0, _EB)], sidx[b], semi[b]).wait()
        pltpu.make_async_copy(dst_hbm.at[pl.ds(0, _EB)], didx[b], semi[b]).wait()
        pltpu.make_async_copy(ea_hbm.at[pl.ds(0, _EB * _R)], eab[b], semi[b]).wait()

    def issue_gather(b, rb):
        pltpu.async_copy(x_hbm.at[sidx[b]], rows[rb], semg[rb])

    def wait_gather(b, rb):
        pltpu.make_async_copy(x_hbm.at[sidx[b]], rows[rb], semg[rb]).wait()

    def issue_scatter(b, rb):
        pltpu.async_copy(rows[rb], zsh.at[didx[b]], sems[rb], add=True)

    def wait_scatter(b, rb):
        pltpu.make_async_copy(rows[rb], zsh.at[didx[b]], sems[rb]).wait()

    def compute(b, rb):
        for g in range(_EB // _L):
            cv = a[0] * eab[b][pl.ds(g * _L, _L)]
            for r in range(1, _R):
                cv = cv + a[r] * eab[b][pl.ds(r * _EB + g * _L, _L)]
            eab[b][pl.ds(g * _L, _L)] = cv

        @plsc.parallel_loop(0, _EB, unroll=4)
        def _scale(e):
            cv = plsc.load_gather(eab[b], [jnp.full((_L,), e, jnp.int32)])
            for j in range(_C // _L):
                rows[rb][e, pl.ds(j * _L, _L)] = cv * rows[rb][e, pl.ds(j * _L, _L)]

    for b in range(3):
        issue_in(b, b)
    wait_in(0)
    issue_gather(0, 0)

    @pl.loop(0, _NB - 2, step=4)
    def _group(g):
        for b in range(4):
            k = g + b
            rb = b % 2
            orb = (b + 1) % 2
            wait_gather(b, rb)
            if b == 0:
                @pl.when(g > 0)
                def _():
                    wait_scatter(3, orb)
            else:
                wait_scatter(b - 1, orb)
            if b == 3:
                @pl.when(g + 6 < _NB)
                def _():
                    issue_in(k + 3, (b + 3) % 4)
            else:
                issue_in(k + 3, (b + 3) % 4)
            wait_in((b + 1) % 4)
            issue_gather((b + 1) % 4, orb)
            compute(b, rb)
            issue_scatter(b, rb)

    wait_gather(0, 0)
    wait_scatter(3, 1)
    wait_in(1)
    issue_gather(1, 1)
    compute(0, 0)
    issue_scatter(0, 0)

    wait_gather(1, 1)
    wait_scatter(0, 0)
    compute(1, 1)
    pltpu.sync_copy(rows[1], zsh.at[didx[1]], add=True)

    plsc.subcore_barrier()

    @pl.when(s < _NS - 1)
    def _out_main():
        pltpu.sync_copy(zsh.at[pl.ds(zoff, _RPT)],
                        out_hbm.at[c, pl.ds(zoff, _RPT)])

    @pl.when(s == _NS - 1)
    def _out_last():
        pltpu.sync_copy(zsh.at[pl.ds(zoff, _RLAST)],
                        out_hbm.at[c, pl.ds(zoff, _RLAST)])


_sc_edge_pass = functools.partial(
    pl.kernel,
    out_type=jax.ShapeDtypeStruct((_NC, _N, _C), jnp.float32),
    mesh=plsc.VectorSubcoreMesh(core_axis_name="c", subcore_axis_name="s"),
    compiler_params=pltpu.CompilerParams(needs_layout_passes=False),
    scratch_types=(
        [pltpu.VMEM((_ZR, _C), jnp.float32),
         pltpu.VMEM((8 * _L,), jnp.float32)]
        + [pltpu.VMEM((_EB,), jnp.int32)] * 4
        + [pltpu.VMEM((_EB,), jnp.int32)] * 4
        + [pltpu.VMEM((_R * _EB,), jnp.float32)] * 4
        + [pltpu.VMEM((_EB, _C), jnp.float32)] * 2
        + [pltpu.VMEM_SHARED((_N, _C), jnp.float32)]
        + [pltpu.SemaphoreType.DMA] * 8
    ),
)(_sc_edge_body)


_TCB = 1000


def _tc_update_body(relu, z_ref, x_ref, w_ref, b_ref, o_ref):
    acc = jnp.dot(z_ref[0], w_ref[0:_C], preferred_element_type=jnp.float32)
    acc = acc + jnp.dot(z_ref[1], w_ref[_C:2 * _C],
                        preferred_element_type=jnp.float32)
    acc = acc + jnp.dot(x_ref[...], w_ref[2 * _C:3 * _C],
                        preferred_element_type=jnp.float32)
    acc = acc + b_ref[...]
    o_ref[...] = jnp.maximum(acc, 0.0) if relu else acc


def _tc_update(z, xin, w, bias, relu):
    body = functools.partial(_tc_update_body, relu)
    return pl.pallas_call(
        body,
        grid=(_N // _TCB,),
        in_specs=[
            pl.BlockSpec((_NC, _TCB, _C), lambda i: (0, i, 0)),
            pl.BlockSpec((_TCB, _C), lambda i: (i, 0)),
            pl.BlockSpec((3 * _C, _C), lambda i: (0, 0)),
            pl.BlockSpec((1, _C), lambda i: (0, 0)),
        ],
        out_specs=pl.BlockSpec((_TCB, _C), lambda i: (i, 0)),
        out_shape=jax.ShapeDtypeStruct((_N, _C), jnp.float32),
    )(z, xin, w, bias)


def kernel(x, edge_index, edge_attr, basis0, att0, root0, bias0,
           basis1, att1, root1, bias1):
    src = edge_index[0]
    dst = edge_index[1]
    ea_pk = edge_attr.reshape(_E // _EB, _EB, _R).transpose(0, 2, 1).reshape(-1)

    h = x
    for basis, att, root, bias, relu in (
            (basis0, att0, root0, bias0, True),
            (basis1, att1, root1, bias1, False)):
        att_pad = jnp.zeros((8, _L), jnp.float32).at[:_R, :_NC].set(att).reshape(-1)
        w = jnp.concatenate([basis[0], basis[1], root], axis=0)
        z = _sc_edge_pass(h, src, dst, ea_pk, att_pad)
        h = _tc_update(z, h, w, bias.reshape(1, _C), relu)
    return h
